# baseline (speedup 1.0000x reference)
import functools

import jax
import jax.numpy as jnp
from jax import lax
from jax.experimental import pallas as pl
from jax.experimental.pallas import tpu as pltpu
from jax.experimental.pallas import tpu_sc as plsc

_DIM = 64
_IDX_LANES = 128
_CHUNK = 512


@functools.cache
def _build_gather(n_total):
    info = plsc.get_sparse_core_info()
    nc, ns = info.num_cores, info.num_subcores
    nw = nc * ns
    rows_per_w = n_total // nw
    n_chunks = rows_per_w // _CHUNK
    k = _CHUNK // _IDX_LANES
    idx_rows_per_w = rows_per_w // _IDX_LANES

    mesh = plsc.VectorSubcoreMesh(core_axis_name="c", subcore_axis_name="s")

    @functools.partial(
        pl.kernel,
        mesh=mesh,
        compiler_params=pltpu.CompilerParams(use_tc_tiling_on_sc=False),
        out_type=jax.ShapeDtypeStruct((n_total, _DIM), jnp.float32),
        scratch_types=[
            pltpu.VMEM((k, _IDX_LANES), jnp.int32),
            pltpu.VMEM((_CHUNK, _DIM), jnp.float32),
            pltpu.SemaphoreType.DMA,
        ],
    )
    def gather_kernel(idx_hbm, table_hbm, out_hbm, idx_v, rows_v, sem):
        wid = lax.axis_index("s") * nc + lax.axis_index("c")
        idx_row0 = wid * idx_rows_per_w
        out_row0 = wid * rows_per_w
        for c in range(n_chunks):
            pltpu.sync_copy(idx_hbm.at[pl.ds(idx_row0 + c * k, k)], idx_v)
            copies = [
                pltpu.async_copy(
                    table_hbm.at[idx_v.at[j]],
                    rows_v.at[pl.ds(j * _IDX_LANES, _IDX_LANES)],
                    sem,
                )
                for j in range(k)
            ]
            for cp in copies:
                cp.wait()
            pltpu.sync_copy(
                rows_v, out_hbm.at[pl.ds(out_row0 + c * _CHUNK, _CHUNK)]
            )

    return gather_kernel


def kernel(x, weight):
    b, f = x.shape
    n_total = b * f
    idx2d = x.reshape(n_total // _IDX_LANES, _IDX_LANES).astype(jnp.int32)
    out = _build_gather(n_total)(idx2d, weight)
    return out.reshape(b, f, _DIM)


# speedup vs baseline: 1.0285x; 1.0285x over previous
import functools

import jax
import jax.numpy as jnp
from jax import lax
from jax.experimental import pallas as pl
from jax.experimental.pallas import tpu as pltpu
from jax.experimental.pallas import tpu_sc as plsc

_DIM = 64
_IDX_LANES = 128
_CHUNK = 512
_NBUF = 3


@functools.cache
def _build_gather(n_total):
    info = plsc.get_sparse_core_info()
    nc, ns = info.num_cores, info.num_subcores
    nw = nc * ns
    rows_per_w = n_total // nw
    n_chunks = rows_per_w // _CHUNK
    k = _CHUNK // _IDX_LANES
    idx_rows_per_w = rows_per_w // _IDX_LANES

    mesh = plsc.VectorSubcoreMesh(core_axis_name="c", subcore_axis_name="s")

    @functools.partial(
        pl.kernel,
        mesh=mesh,
        compiler_params=pltpu.CompilerParams(use_tc_tiling_on_sc=False),
        out_type=jax.ShapeDtypeStruct((n_total, _DIM), jnp.float32),
        scratch_types=[
            pltpu.VMEM((idx_rows_per_w, _IDX_LANES), jnp.int32),
            pltpu.VMEM((_NBUF, _CHUNK, _DIM), jnp.float32),
            pltpu.SemaphoreType.DMA((_NBUF,)),
            pltpu.SemaphoreType.DMA((_NBUF,)),
        ],
    )
    def gather_kernel(idx_hbm, table_hbm, out_hbm, idx_all, rows_v, gsem, ssem):
        wid = lax.axis_index("s") * nc + lax.axis_index("c")
        idx_row0 = wid * idx_rows_per_w
        out_row0 = wid * rows_per_w
        pltpu.sync_copy(idx_hbm.at[pl.ds(idx_row0, idx_rows_per_w)], idx_all)

        def start_gathers(c, b):
            return [
                pltpu.async_copy(
                    table_hbm.at[idx_all.at[c * k + j]],
                    rows_v.at[b, pl.ds(j * _IDX_LANES, _IDX_LANES)],
                    gsem.at[b],
                )
                for j in range(k)
            ]

        def start_store(c, b):
            return pltpu.async_copy(
                rows_v.at[b],
                out_hbm.at[pl.ds(out_row0 + c * _CHUNK, _CHUNK)],
                ssem.at[b],
            )

        stores = {}
        pend = {}
        for c in range(n_chunks):
            b = c % _NBUF
            if c >= _NBUF:
                stores.pop(b).wait()
            pend[b] = start_gathers(c, b)
            if c >= 1:
                bp = (c - 1) % _NBUF
                for cp in pend.pop(bp):
                    cp.wait()
                stores[bp] = start_store(c - 1, bp)
        blast = (n_chunks - 1) % _NBUF
        for cp in pend.pop(blast):
            cp.wait()
        stores[blast] = start_store(n_chunks - 1, blast)
        for b in list(stores):
            stores.pop(b).wait()

    return gather_kernel


def kernel(x, weight):
    b, f = x.shape
    n_total = b * f
    idx2d = x.reshape(n_total // _IDX_LANES, _IDX_LANES).astype(jnp.int32)
    out = _build_gather(n_total)(idx2d, weight)
    return out.reshape(b, f, _DIM)
